# baseline (speedup 1.0000x reference)
import functools

import jax
import jax.numpy as jnp
from jax.experimental import pallas as pl
from jax.experimental.pallas import tpu as pltpu

_K = 16
_FOLD = 64


def _merge22(lo_a, hi_a, lo_b, hi_b):
    s1 = jnp.minimum(lo_a, lo_b)
    s4 = jnp.maximum(hi_a, hi_b)
    t1 = jnp.maximum(lo_a, lo_b)
    t2 = jnp.minimum(hi_a, hi_b)
    return s1, jnp.minimum(t1, t2), jnp.maximum(t1, t2), s4


def _low4_of_sorted44(a, b):
    l1 = jnp.minimum(a[0], b[3])
    l2 = jnp.minimum(a[1], b[2])
    l3 = jnp.minimum(a[2], b[1])
    l4 = jnp.minimum(a[3], b[0])
    m1, m3 = jnp.minimum(l1, l3), jnp.maximum(l1, l3)
    m2, m4 = jnp.minimum(l2, l4), jnp.maximum(l2, l4)
    return (
        jnp.minimum(m1, m2),
        jnp.maximum(m1, m2),
        jnp.minimum(m3, m4),
        jnp.maximum(m3, m4),
    )


def _topk_body(rows, n, k, p_rows_ref, p_all_ref, out_ref,
               hcol_ref, f1_ref, f2_ref, f3_ref, f4_ref):
    i = pl.program_id(0)
    blocks_per_graph = n // rows
    rb = i % blocks_per_graph
    s_width = n // _FOLD

    @pl.when(rb == 0)
    def _():
        pa = p_all_ref[...]
        hcol_ref[...] = 0.5 * jnp.sum(pa * pa, axis=1)[None, :]

    p_rows = p_rows_ref[...]
    hrow = 0.5 * jnp.sum(p_rows * p_rows, axis=1, keepdims=True)
    dots = jax.lax.dot_general(
        p_rows, p_all_ref[...], (((1,), (1,)), ((), ())),
        preferred_element_type=jnp.float32,
    )
    d2 = (hrow + hcol_ref[...]) - dots

    bits = jax.lax.bitcast_convert_type(d2, jnp.int32)
    pk = []
    for c in range(_FOLD):
        chunk = bits[:, c * s_width:(c + 1) * s_width]
        chunk = jax.lax.bitwise_or(
            jax.lax.bitwise_and(chunk, jnp.int32(~(_FOLD - 1))), jnp.int32(c)
        )
        pk.append(jax.lax.bitcast_convert_type(chunk, jnp.float32))

    lo = [jnp.minimum(pk[2 * t], pk[2 * t + 1]) for t in range(_FOLD // 2)]
    hi = [jnp.maximum(pk[2 * t], pk[2 * t + 1]) for t in range(_FOLD // 2)]
    s4 = [
        _merge22(lo[2 * t], hi[2 * t], lo[2 * t + 1], hi[2 * t + 1])
        for t in range(_FOLD // 4)
    ]
    while len(s4) > 1:
        s4 = [
            _low4_of_sorted44(s4[2 * t], s4[2 * t + 1])
            for t in range(len(s4) // 2)
        ]
    f1, f2, f3, f4 = s4[0]
    f1_ref[...] = f1
    f2_ref[...] = f2
    f3_ref[...] = f3
    f4_ref[...] = f4

    slotf = jax.lax.broadcasted_iota(
        jnp.int32, (rows, s_width), 1).astype(jnp.float32)
    inf = jnp.float32(jnp.inf)
    for j in range(k + 1):
        f1 = f1_ref[...]
        m = jnp.min(f1, axis=1, keepdims=True)
        sf = jnp.min(jnp.where(f1 == m, slotf, inf), axis=1, keepdims=True)
        if j >= 1:
            c = jax.lax.bitwise_and(
                jax.lax.bitcast_convert_type(m, jnp.int32),
                jnp.int32(_FOLD - 1),
            )
            out_ref[:, j - 1:j] = c * s_width + sf.astype(jnp.int32)
        if j < k:
            pred = slotf == sf
            f2v = f2_ref[...]
            f3v = f3_ref[...]
            f4v = f4_ref[...]
            f1_ref[...] = jnp.where(pred, f2v, f1)
            f2_ref[...] = jnp.where(pred, f3v, f2v)
            f3_ref[...] = jnp.where(pred, f4v, f3v)
            f4_ref[...] = jnp.where(pred, inf, f4v)


def _neighbors(positions, num_graphs, n, k):
    total = positions.shape[0]
    rows = min(512, n)
    blocks_per_graph = n // rows
    s_width = n // _FOLD
    grid = (num_graphs * blocks_per_graph,)
    body = functools.partial(_topk_body, rows, n, k)
    return pl.pallas_call(
        body,
        grid=grid,
        in_specs=[
            pl.BlockSpec((rows, positions.shape[1]), lambda i: (i, 0)),
            pl.BlockSpec(
                (n, positions.shape[1]),
                lambda i, _bpg=blocks_per_graph: (i // _bpg, 0),
            ),
        ],
        out_specs=pl.BlockSpec((rows, k), lambda i: (i, 0)),
        out_shape=jax.ShapeDtypeStruct((total, k), jnp.int32),
        scratch_shapes=[pltpu.VMEM((1, n), jnp.float32)]
        + [pltpu.VMEM((rows, s_width), jnp.float32)] * 4,
    )(positions, positions)


def kernel(num_nodes, positions):
    num_graphs = num_nodes.shape[0]
    total = positions.shape[0]
    n = total // num_graphs
    k = min(_K, n - 1)

    idx_local = _neighbors(positions, num_graphs, n, k)

    offsets = jnp.concatenate(
        (jnp.zeros((1,), dtype=num_nodes.dtype), jnp.cumsum(num_nodes)[:-1])
    ).astype(jnp.int32)
    row_off = jnp.repeat(offsets, n)
    idx_to = idx_local + row_off[:, None]
    idx_from = (jnp.arange(n, dtype=jnp.int32)[None, :] + offsets[:, None]).reshape(-1)
    edge_index = jnp.stack(
        (jnp.repeat(idx_from, k), idx_to.reshape(-1)), axis=-1
    )
    num_edges = jnp.full((num_graphs,), n * k, dtype=jnp.int32)
    return edge_index, num_edges


# speedup vs baseline: 1.4539x; 1.4539x over previous
import functools

import jax
import jax.numpy as jnp
from jax.experimental import pallas as pl
from jax.experimental.pallas import tpu as pltpu

_K = 16
_FOLD = 32


def _merge22(lo_a, hi_a, lo_b, hi_b):
    s1 = jnp.minimum(lo_a, lo_b)
    s4 = jnp.maximum(hi_a, hi_b)
    t1 = jnp.maximum(lo_a, lo_b)
    t2 = jnp.minimum(hi_a, hi_b)
    return s1, jnp.minimum(t1, t2), jnp.maximum(t1, t2), s4


def _low4_of_sorted44(a, b):
    l1 = jnp.minimum(a[0], b[3])
    l2 = jnp.minimum(a[1], b[2])
    l3 = jnp.minimum(a[2], b[1])
    l4 = jnp.minimum(a[3], b[0])
    m1, m3 = jnp.minimum(l1, l3), jnp.maximum(l1, l3)
    m2, m4 = jnp.minimum(l2, l4), jnp.maximum(l2, l4)
    return (
        jnp.minimum(m1, m2),
        jnp.maximum(m1, m2),
        jnp.minimum(m3, m4),
        jnp.maximum(m3, m4),
    )


def _topk_body(rows, n, k, p_rows_ref, p_all_ref, out_ref,
               hcol_ref, f1_ref, f2_ref, f3_ref, f4_ref):
    i = pl.program_id(0)
    blocks_per_graph = n // rows
    rb = i % blocks_per_graph
    s_width = n // _FOLD

    @pl.when(rb == 0)
    def _():
        pa = p_all_ref[...]
        hcol_ref[...] = 0.5 * jnp.sum(pa * pa, axis=1)[None, :]

    p_rows = p_rows_ref[...]
    hrow = 0.5 * jnp.sum(p_rows * p_rows, axis=1, keepdims=True)
    dots = jax.lax.dot_general(
        p_rows, p_all_ref[...], (((1,), (1,)), ((), ())),
        preferred_element_type=jnp.float32,
    )
    d2 = (hrow + hcol_ref[...]) - dots

    bits = jax.lax.bitcast_convert_type(d2, jnp.int32)
    pk = []
    for c in range(_FOLD):
        chunk = bits[:, c * s_width:(c + 1) * s_width]
        chunk = jax.lax.bitwise_or(
            jax.lax.bitwise_and(chunk, jnp.int32(~(_FOLD - 1))), jnp.int32(c)
        )
        pk.append(jax.lax.bitcast_convert_type(chunk, jnp.float32))

    lo = [jnp.minimum(pk[2 * t], pk[2 * t + 1]) for t in range(_FOLD // 2)]
    hi = [jnp.maximum(pk[2 * t], pk[2 * t + 1]) for t in range(_FOLD // 2)]
    s4 = [
        _merge22(lo[2 * t], hi[2 * t], lo[2 * t + 1], hi[2 * t + 1])
        for t in range(_FOLD // 4)
    ]
    while len(s4) > 1:
        s4 = [
            _low4_of_sorted44(s4[2 * t], s4[2 * t + 1])
            for t in range(len(s4) // 2)
        ]
    f1, f2, f3, f4 = s4[0]
    f1_ref[...] = f1
    f2_ref[...] = f2
    f3_ref[...] = f3
    f4_ref[...] = f4

    slotf = jax.lax.broadcasted_iota(
        jnp.int32, (rows, s_width), 1).astype(jnp.float32)
    inf = jnp.float32(jnp.inf)
    for j in range(k + 1):
        f1 = f1_ref[...]
        m = jnp.min(f1, axis=1, keepdims=True)
        sf = jnp.min(jnp.where(f1 == m, slotf, inf), axis=1, keepdims=True)
        if j >= 1:
            c = jax.lax.bitwise_and(
                jax.lax.bitcast_convert_type(m, jnp.int32),
                jnp.int32(_FOLD - 1),
            )
            out_ref[:, j - 1:j] = c * s_width + sf.astype(jnp.int32)
        if j < k:
            pred = slotf == sf
            f2v = f2_ref[...]
            f3v = f3_ref[...]
            f4v = f4_ref[...]
            f1_ref[...] = jnp.where(pred, f2v, f1)
            f2_ref[...] = jnp.where(pred, f3v, f2v)
            f3_ref[...] = jnp.where(pred, f4v, f3v)
            f4_ref[...] = jnp.where(pred, inf, f4v)


def _neighbors(positions, num_graphs, n, k):
    total = positions.shape[0]
    rows = min(512, n)
    blocks_per_graph = n // rows
    s_width = n // _FOLD
    grid = (num_graphs * blocks_per_graph,)
    body = functools.partial(_topk_body, rows, n, k)
    return pl.pallas_call(
        body,
        grid=grid,
        in_specs=[
            pl.BlockSpec((rows, positions.shape[1]), lambda i: (i, 0)),
            pl.BlockSpec(
                (n, positions.shape[1]),
                lambda i, _bpg=blocks_per_graph: (i // _bpg, 0),
            ),
        ],
        out_specs=pl.BlockSpec((rows, k), lambda i: (i, 0)),
        out_shape=jax.ShapeDtypeStruct((total, k), jnp.int32),
        scratch_shapes=[pltpu.VMEM((1, n), jnp.float32)]
        + [pltpu.VMEM((rows, s_width), jnp.float32)] * 4,
    )(positions, positions)


def kernel(num_nodes, positions):
    num_graphs = num_nodes.shape[0]
    total = positions.shape[0]
    n = total // num_graphs
    k = min(_K, n - 1)

    idx_local = _neighbors(positions, num_graphs, n, k)

    offsets = jnp.concatenate(
        (jnp.zeros((1,), dtype=num_nodes.dtype), jnp.cumsum(num_nodes)[:-1])
    ).astype(jnp.int32)
    row_off = jnp.repeat(offsets, n)
    idx_to = idx_local + row_off[:, None]
    idx_from = (jnp.arange(n, dtype=jnp.int32)[None, :] + offsets[:, None]).reshape(-1)
    edge_index = jnp.stack(
        (jnp.repeat(idx_from, k), idx_to.reshape(-1)), axis=-1
    )
    num_edges = jnp.full((num_graphs,), n * k, dtype=jnp.int32)
    return edge_index, num_edges
